# baseline (speedup 1.0000x reference)
import functools

import jax
import jax.numpy as jnp
from jax import lax
from jax.experimental import pallas as pl
from jax.experimental.pallas import tpu as pltpu
from jax.experimental.pallas import tpu_sc as plsc

_VOCAB = 800
_EMB = 128
_BATCH = 16384

_NC = 2
_NS = 16
_NW = _NC * _NS
_L = 16
_BPW = _BATCH // _NW


def _tc_scores_body(table_ref, w_ref, b_ref, t_ref):
    t = jax.lax.dot_general(
        w_ref[...], table_ref[...],
        dimension_numbers=(((1,), (1,)), ((), ())),
        preferred_element_type=jnp.float32)
    rowid = jax.lax.broadcasted_iota(jnp.int32, (2, _VOCAB), 0)
    t_ref[...] = t + jnp.where(rowid == 0, b_ref[0], 0.0)


def _tc_scores(table, w2row, b):
    return pl.pallas_call(
        _tc_scores_body,
        out_shape=jax.ShapeDtypeStruct((2, _VOCAB), jnp.float32),
        in_specs=[
            pl.BlockSpec(memory_space=pltpu.VMEM),
            pl.BlockSpec(memory_space=pltpu.VMEM),
            pl.BlockSpec(memory_space=pltpu.SMEM),
        ],
    )(table, w2row, b)


def _sc_gather_body(t_hbm, first_hbm, second_hbm, out_hbm,
                    t_v, f_v, s_v, o_v, o_sh, sem):
    sid = lax.axis_index("s")
    cid = lax.axis_index("c")
    wid = cid * _NS + sid
    base = wid * _BPW
    c1 = pltpu.async_copy(t_hbm, t_v, sem)
    c3 = pltpu.async_copy(first_hbm.at[pl.ds(base, _BPW)], f_v, sem)
    c4 = pltpu.async_copy(second_hbm.at[pl.ds(base, _BPW)], s_v, sem)
    c1.wait()
    c3.wait()
    c4.wait()

    row0 = lax.iota(jnp.int32, _L) * 0
    row1 = row0 + 1

    @plsc.parallel_loop(0, _BPW, _L, unroll=2)
    def _gather_step(off):
        a = plsc.load_gather(t_v, [row0, f_v[pl.ds(off, _L)]])
        c = plsc.load_gather(t_v, [row1, s_v[pl.ds(off, _L)]])
        x = a + c
        o_v[pl.ds(off, _L)] = 1.0 / (1.0 + jnp.exp(-x))

    pltpu.sync_copy(o_v, o_sh.at[pl.ds(sid * _BPW, _BPW)])
    plsc.subcore_barrier()

    @pl.when(sid == 0)
    def _():
        pltpu.sync_copy(o_sh, out_hbm.at[pl.ds(cid * _NS * _BPW, _NS * _BPW)])


_sc_gather = functools.partial(
    pl.kernel,
    out_type=jax.ShapeDtypeStruct((_BATCH,), jnp.float32),
    mesh=plsc.VectorSubcoreMesh(core_axis_name="c", subcore_axis_name="s"),
    compiler_params=pltpu.CompilerParams(
        needs_layout_passes=False, skip_device_barrier=True),
    scratch_types=[
        pltpu.VMEM((2, _VOCAB), jnp.float32),
        pltpu.VMEM((_BPW,), jnp.int32),
        pltpu.VMEM((_BPW,), jnp.int32),
        pltpu.VMEM((_BPW,), jnp.float32),
        pltpu.VMEM_SHARED((_NS * _BPW,), jnp.float32),
        pltpu.SemaphoreType.DMA,
    ],
)(_sc_gather_body)


@jax.jit
def kernel(first, second, table, W, b):
    w2row = W.reshape(2, _EMB)
    t = _tc_scores(table, w2row, b)
    out = _sc_gather(t, first.astype(jnp.int32), second.astype(jnp.int32))
    return out.reshape(_BATCH, 1)


# speedup vs baseline: 1.0144x; 1.0144x over previous
import functools

import jax
import jax.numpy as jnp
from jax import lax
from jax.experimental import pallas as pl
from jax.experimental.pallas import tpu as pltpu
from jax.experimental.pallas import tpu_sc as plsc

_VOCAB = 800
_EMB = 128
_BATCH = 16384

_NC = 2
_NS = 16
_NW = _NC * _NS
_L = 16
_BPW = _BATCH // _NW


def _tc_scores_body(table_ref, w_ref, b_ref, t_ref):
    t = jax.lax.dot_general(
        w_ref[...], table_ref[...],
        dimension_numbers=(((1,), (1,)), ((), ())),
        preferred_element_type=jnp.float32)
    rowid = jax.lax.broadcasted_iota(jnp.int32, (2, _VOCAB), 0)
    t_ref[...] = t + jnp.where(rowid == 0, b_ref[0], 0.0)


def _tc_scores(table, w2row, b):
    return pl.pallas_call(
        _tc_scores_body,
        out_shape=jax.ShapeDtypeStruct((2, _VOCAB), jnp.float32),
        in_specs=[
            pl.BlockSpec(memory_space=pltpu.VMEM),
            pl.BlockSpec(memory_space=pltpu.VMEM),
            pl.BlockSpec(memory_space=pltpu.SMEM),
        ],
    )(table, w2row, b)


def _sc_gather_body(t_hbm, first_hbm, second_hbm, out_hbm,
                    t_v, f_v, s_v, o_v, sem):
    sid = lax.axis_index("s")
    cid = lax.axis_index("c")
    wid = cid * _NS + sid
    base = wid * _BPW
    c1 = pltpu.async_copy(t_hbm, t_v, sem)
    c3 = pltpu.async_copy(first_hbm.at[pl.ds(base, _BPW)], f_v, sem)
    c4 = pltpu.async_copy(second_hbm.at[pl.ds(base, _BPW)], s_v, sem)
    c1.wait()
    c3.wait()
    c4.wait()

    row0 = lax.iota(jnp.int32, _L) * 0
    row1 = row0 + 1

    @plsc.parallel_loop(0, _BPW, _L, unroll=2)
    def _gather_step(off):
        a = plsc.load_gather(t_v, [row0, f_v[pl.ds(off, _L)]])
        c = plsc.load_gather(t_v, [row1, s_v[pl.ds(off, _L)]])
        x = a + c
        o_v[pl.ds(off, _L)] = 1.0 / (1.0 + jnp.exp(-x))

    pltpu.sync_copy(o_v, out_hbm.at[pl.ds(base, _BPW)])


_sc_gather = functools.partial(
    pl.kernel,
    out_type=jax.ShapeDtypeStruct((_BATCH,), jnp.float32),
    mesh=plsc.VectorSubcoreMesh(core_axis_name="c", subcore_axis_name="s"),
    compiler_params=pltpu.CompilerParams(
        needs_layout_passes=False, skip_device_barrier=True),
    scratch_types=[
        pltpu.VMEM((2, _VOCAB), jnp.float32),
        pltpu.VMEM((_BPW,), jnp.int32),
        pltpu.VMEM((_BPW,), jnp.int32),
        pltpu.VMEM((_BPW,), jnp.float32),
        pltpu.SemaphoreType.DMA,
    ],
)(_sc_gather_body)


@jax.jit
def kernel(first, second, table, W, b):
    w2row = W.reshape(2, _EMB)
    t = _tc_scores(table, w2row, b)
    out = _sc_gather(t, first.astype(jnp.int32), second.astype(jnp.int32))
    return out.reshape(_BATCH, 1)
